# baseline (speedup 1.0000x reference)
import functools

import jax
import jax.numpy as jnp
from jax import lax
from jax.experimental import pallas as pl
from jax.experimental.pallas import tpu as pltpu
from jax.experimental.pallas import tpu_sc as plsc

NUM_HEADS = 16
NEG = float(jnp.finfo(jnp.float32).min)
NSTREAM = 16


def _mask_body(dec_mask_ref, enc_mask_ref, dec_out_ref, enc_out_ref):
    i = pl.program_id(0)
    _, _, R, S = dec_out_ref.shape
    row = i * R + lax.broadcasted_iota(jnp.int32, (1, 1, R, S), 2)
    col = lax.broadcasted_iota(jnp.int32, (1, 1, R, S), 3)
    causal = jnp.where(col <= row, 1.0, 0.0)
    m = dec_mask_ref[0, :].astype(jnp.float32)[None, None, None, :]
    dec_out_ref[...] = (1.0 - causal * m) * NEG
    e = enc_mask_ref[0, :].astype(jnp.float32)[None, None, None, :]
    enc_out_ref[...] = (1.0 - e) * NEG


def _make_masks(dec_mask, enc_mask):
    _, s_dec = dec_mask.shape
    _, s_enc = enc_mask.shape
    rows_per_step = 256
    grid = s_dec // rows_per_step
    return pl.pallas_call(
        _mask_body,
        grid=(grid,),
        in_specs=[
            pl.BlockSpec((1, s_dec), lambda i: (0, 0)),
            pl.BlockSpec((1, s_enc), lambda i: (0, 0)),
        ],
        out_specs=[
            pl.BlockSpec((1, 1, rows_per_step, s_dec), lambda i: (0, 0, i, 0)),
            pl.BlockSpec((1, 1, 1, s_enc), lambda i: (0, 0, 0, 0)),
        ],
        out_shape=[
            jax.ShapeDtypeStruct((1, 1, s_dec, s_dec), jnp.float32),
            jax.ShapeDtypeStruct((1, 1, 1, s_enc), jnp.float32),
        ],
    )(dec_mask, enc_mask)


def _copy_body(epb_ref, ehs_ref, epb_out_ref, ehs_out_ref, sems, sem_ehs):
    n = epb_ref.shape[0]
    chunk = n // NSTREAM
    cps = [
        pltpu.make_async_copy(epb_ref.at[pl.ds(j * chunk, chunk)],
                              epb_out_ref.at[pl.ds(j * chunk, chunk)],
                              sems.at[j])
        for j in range(NSTREAM)
    ]
    cp_ehs = pltpu.make_async_copy(ehs_ref, ehs_out_ref, sem_ehs)
    for cp in cps:
        cp.start()
    cp_ehs.start()
    for cp in cps:
        cp.wait()
    cp_ehs.wait()


def _copy_passthrough(epb_flat, ehs):
    return pl.pallas_call(
        _copy_body,
        in_specs=[
            pl.BlockSpec(memory_space=pl.ANY),
            pl.BlockSpec(memory_space=pl.ANY),
        ],
        out_specs=[
            pl.BlockSpec(memory_space=pl.ANY),
            pl.BlockSpec(memory_space=pl.ANY),
        ],
        out_shape=[
            jax.ShapeDtypeStruct(epb_flat.shape, jnp.float32),
            jax.ShapeDtypeStruct(ehs.shape, jnp.float32),
        ],
        scratch_shapes=[
            pltpu.SemaphoreType.DMA((NSTREAM,)),
            pltpu.SemaphoreType.DMA,
        ],
    )(epb_flat, ehs)


@functools.lru_cache(maxsize=None)
def _make_sc_gather(n_tok, d_model):
    info = plsc.get_sparse_core_info()
    nc, ns = info.num_cores, info.num_subcores
    nw = nc * ns
    bpw = n_tok // nw
    mesh = plsc.VectorSubcoreMesh(core_axis_name="c", subcore_axis_name="s")

    @functools.partial(
        pl.kernel,
        mesh=mesh,
        out_type=jax.ShapeDtypeStruct((n_tok, d_model), jnp.float32),
        scratch_types=[
            pltpu.VMEM((bpw,), jnp.int32),
            pltpu.VMEM((bpw, d_model), jnp.float32),
            pltpu.SemaphoreType.DMA,
        ],
    )
    def gather_k(table_hbm, idx_hbm, out_hbm, idx_v, rows_v, sem):
        wid = lax.axis_index("s") * nc + lax.axis_index("c")
        base = wid * bpw
        pltpu.sync_copy(idx_hbm.at[pl.ds(base, bpw)], idx_v)
        pltpu.async_copy(table_hbm.at[idx_v], rows_v, sem).wait()
        pltpu.sync_copy(rows_v, out_hbm.at[pl.ds(base, bpw)])

    return gather_k


def kernel(encoder_hidden_states, encoder_position_bias, decoder_input_ids,
           decoder_attention_mask, encoder_attention_mask, embedding_weight):
    b, s_dec = decoder_input_ids.shape
    vocab, d_model = embedding_weight.shape
    ids_flat = decoder_input_ids.reshape(-1)
    _, nh, s_q, s_k = encoder_position_bias.shape
    epb_flat = encoder_position_bias.reshape(b * nh * s_q, s_k)

    gather_k = _make_sc_gather(b * s_dec, d_model)
    decoder_hidden_states = gather_k(embedding_weight, ids_flat)
    decoder_hidden_states = decoder_hidden_states.reshape(b, s_dec, d_model)

    epb_out, ehs_out = _copy_passthrough(epb_flat, encoder_hidden_states)
    epb_out = epb_out.reshape(encoder_position_bias.shape)

    dec_ext, enc_ext = _make_masks(decoder_attention_mask, encoder_attention_mask)

    decoder_position_bias = jnp.zeros((b, NUM_HEADS, s_dec, 1), dtype=jnp.float32)

    return (ehs_out, epb_out, decoder_hidden_states,
            enc_ext, dec_ext, decoder_position_bias)


# speedup vs baseline: 40.7194x; 40.7194x over previous
import functools

import jax
import jax.numpy as jnp
from jax import lax
from jax.experimental import pallas as pl
from jax.experimental.pallas import tpu as pltpu
from jax.experimental.pallas import tpu_sc as plsc

NUM_HEADS = 16
NEG = float(jnp.finfo(jnp.float32).min)
NSTREAM = 16


def _mask_body(dec_mask_ref, enc_mask_ref, dec_out_ref, enc_out_ref):
    i = pl.program_id(0)
    _, _, R, S = dec_out_ref.shape
    row = i * R + lax.broadcasted_iota(jnp.int32, (1, 1, R, S), 2)
    col = lax.broadcasted_iota(jnp.int32, (1, 1, R, S), 3)
    causal = jnp.where(col <= row, 1.0, 0.0)
    m = dec_mask_ref[0, :].astype(jnp.float32)[None, None, None, :]
    dec_out_ref[...] = (1.0 - causal * m) * NEG
    e = enc_mask_ref[0, :].astype(jnp.float32)[None, None, None, :]
    enc_out_ref[...] = (1.0 - e) * NEG


def _make_masks(dec_mask, enc_mask):
    _, s_dec = dec_mask.shape
    _, s_enc = enc_mask.shape
    rows_per_step = 256
    grid = s_dec // rows_per_step
    return pl.pallas_call(
        _mask_body,
        grid=(grid,),
        in_specs=[
            pl.BlockSpec((1, s_dec), lambda i: (0, 0)),
            pl.BlockSpec((1, s_enc), lambda i: (0, 0)),
        ],
        out_specs=[
            pl.BlockSpec((1, 1, rows_per_step, s_dec), lambda i: (0, 0, i, 0)),
            pl.BlockSpec((1, 1, 1, s_enc), lambda i: (0, 0, 0, 0)),
        ],
        out_shape=[
            jax.ShapeDtypeStruct((1, 1, s_dec, s_dec), jnp.float32),
            jax.ShapeDtypeStruct((1, 1, 1, s_enc), jnp.float32),
        ],
    )(dec_mask, enc_mask)


def _copy_body(src_ref, dst_ref):
    dst_ref[...] = src_ref[...]


def _copy_passthrough(epb_flat):
    n, w = epb_flat.shape
    rows = 512
    return pl.pallas_call(
        _copy_body,
        grid=(n // rows,),
        in_specs=[pl.BlockSpec((rows, w), lambda i: (i, 0))],
        out_specs=pl.BlockSpec((rows, w), lambda i: (i, 0)),
        out_shape=jax.ShapeDtypeStruct(epb_flat.shape, jnp.float32),
    )(epb_flat)


@functools.lru_cache(maxsize=None)
def _make_sc_gather(n_tok, d_model):
    info = plsc.get_sparse_core_info()
    nc, ns = info.num_cores, info.num_subcores
    nw = nc * ns
    bpw = n_tok // nw
    mesh = plsc.VectorSubcoreMesh(core_axis_name="c", subcore_axis_name="s")

    @functools.partial(
        pl.kernel,
        mesh=mesh,
        out_type=jax.ShapeDtypeStruct((n_tok, d_model), jnp.float32),
        scratch_types=[
            pltpu.VMEM((bpw,), jnp.int32),
            pltpu.VMEM((bpw, d_model), jnp.float32),
            pltpu.SemaphoreType.DMA,
        ],
    )
    def gather_k(table_hbm, idx_hbm, out_hbm, idx_v, rows_v, sem):
        wid = lax.axis_index("s") * nc + lax.axis_index("c")
        base = wid * bpw
        pltpu.sync_copy(idx_hbm.at[pl.ds(base, bpw)], idx_v)
        pltpu.async_copy(table_hbm.at[idx_v], rows_v, sem).wait()
        pltpu.sync_copy(rows_v, out_hbm.at[pl.ds(base, bpw)])

    return gather_k


def kernel(encoder_hidden_states, encoder_position_bias, decoder_input_ids,
           decoder_attention_mask, encoder_attention_mask, embedding_weight):
    b, s_dec = decoder_input_ids.shape
    vocab, d_model = embedding_weight.shape
    ids_flat = decoder_input_ids.reshape(-1)
    _, nh, s_q, s_k = encoder_position_bias.shape
    epb_flat = encoder_position_bias.reshape(b * nh * s_q, s_k)

    gather_k = _make_sc_gather(b * s_dec, d_model)
    decoder_hidden_states = gather_k(embedding_weight, ids_flat)
    decoder_hidden_states = decoder_hidden_states.reshape(b, s_dec, d_model)

    epb_out = _copy_passthrough(epb_flat).reshape(encoder_position_bias.shape)
    ehs_out = encoder_hidden_states

    dec_ext, enc_ext = _make_masks(decoder_attention_mask, encoder_attention_mask)

    decoder_position_bias = jnp.zeros((b, NUM_HEADS, s_dec, 1), dtype=jnp.float32)

    return (ehs_out, epb_out, decoder_hidden_states,
            enc_ext, dec_ext, decoder_position_bias)


# speedup vs baseline: 41.7391x; 1.0250x over previous
import functools

import jax
import jax.numpy as jnp
from jax import lax
from jax.experimental import pallas as pl
from jax.experimental.pallas import tpu as pltpu
from jax.experimental.pallas import tpu_sc as plsc

NUM_HEADS = 16
NEG = float(jnp.finfo(jnp.float32).min)
NSTREAM = 16


def _mask_body(dec_mask_ref, enc_mask_ref, dec_out_ref, enc_out_ref):
    i = pl.program_id(0)
    _, _, R, S = dec_out_ref.shape
    row = i * R + lax.broadcasted_iota(jnp.int32, (1, 1, R, S), 2)
    col = lax.broadcasted_iota(jnp.int32, (1, 1, R, S), 3)
    causal = jnp.where(col <= row, 1.0, 0.0)
    m = dec_mask_ref[0, :].astype(jnp.float32)[None, None, None, :]
    dec_out_ref[...] = (1.0 - causal * m) * NEG
    e = enc_mask_ref[0, :].astype(jnp.float32)[None, None, None, :]
    enc_out_ref[...] = (1.0 - e) * NEG


def _make_masks(dec_mask, enc_mask):
    _, s_dec = dec_mask.shape
    _, s_enc = enc_mask.shape
    rows_per_step = 256
    grid = s_dec // rows_per_step
    return pl.pallas_call(
        _mask_body,
        grid=(grid,),
        in_specs=[
            pl.BlockSpec((1, s_dec), lambda i: (0, 0)),
            pl.BlockSpec((1, s_enc), lambda i: (0, 0)),
        ],
        out_specs=[
            pl.BlockSpec((1, 1, rows_per_step, s_dec), lambda i: (0, 0, i, 0)),
            pl.BlockSpec((1, 1, 1, s_enc), lambda i: (0, 0, 0, 0)),
        ],
        out_shape=[
            jax.ShapeDtypeStruct((1, 1, s_dec, s_dec), jnp.float32),
            jax.ShapeDtypeStruct((1, 1, 1, s_enc), jnp.float32),
        ],
    )(dec_mask, enc_mask)


@functools.lru_cache(maxsize=None)
def _make_sc_gather(n_tok, d_model, n_ehs):
    info = plsc.get_sparse_core_info()
    nc, ns = info.num_cores, info.num_subcores
    nw = nc * ns
    bpw = n_tok // nw
    epw = n_ehs // nw
    nchunk = 4
    eh = epw // nchunk
    mesh = plsc.VectorSubcoreMesh(core_axis_name="c", subcore_axis_name="s")

    @functools.partial(
        pl.kernel,
        mesh=mesh,
        out_type=(
            jax.ShapeDtypeStruct((n_tok, d_model), jnp.float32),
            jax.ShapeDtypeStruct((n_ehs, d_model), jnp.float32),
        ),
        scratch_types=[
            pltpu.VMEM((bpw,), jnp.int32),
            pltpu.VMEM((bpw, d_model), jnp.float32),
            pltpu.VMEM((2, eh, d_model), jnp.float32),
            pltpu.SemaphoreType.DMA,
            pltpu.SemaphoreType.DMA,
            pltpu.SemaphoreType.DMA((2,)),
            pltpu.SemaphoreType.DMA((2,)),
        ],
    )
    def gather_k(table_hbm, idx_hbm, ehs_hbm, hid_out, ehs_out,
                 idx_v, rows_v, ebuf, sem_g, sem_go, sem_ei, sem_eo):
        wid = lax.axis_index("s") * nc + lax.axis_index("c")
        base = wid * bpw
        ebase = wid * epw

        def cin(ci, slot):
            return pltpu.async_copy(
                ehs_hbm.at[pl.ds(ebase + ci * eh, eh)], ebuf.at[slot],
                sem_ei.at[slot])

        def cout(ci, slot):
            return pltpu.async_copy(
                ebuf.at[slot], ehs_out.at[pl.ds(ebase + ci * eh, eh)],
                sem_eo.at[slot])

        ein = [None] * nchunk
        eout = [None] * nchunk
        ein[0] = cin(0, 0)
        ein[1] = cin(1, 1)
        pltpu.sync_copy(idx_hbm.at[pl.ds(base, bpw)], idx_v)
        g = pltpu.async_copy(table_hbm.at[idx_v], rows_v, sem_g)
        for ci in range(nchunk):
            slot = ci % 2
            ein[ci].wait()
            eout[ci] = cout(ci, slot)
            if ci + 2 < nchunk:
                eout[ci].wait()
                ein[ci + 2] = cin(ci + 2, slot)
        g.wait()
        go = pltpu.async_copy(rows_v, hid_out.at[pl.ds(base, bpw)], sem_go)
        eout[nchunk - 2].wait()
        eout[nchunk - 1].wait()
        go.wait()

    return gather_k


def kernel(encoder_hidden_states, encoder_position_bias, decoder_input_ids,
           decoder_attention_mask, encoder_attention_mask, embedding_weight):
    b, s_dec = decoder_input_ids.shape
    vocab, d_model = embedding_weight.shape
    ids_flat = decoder_input_ids.reshape(-1)
    _, nh, s_q, s_k = encoder_position_bias.shape
    epb_flat = encoder_position_bias.reshape(b * nh * s_q, s_k)

    _, s_enc, _ = encoder_hidden_states.shape
    ehs_flat = encoder_hidden_states.reshape(b * s_enc, d_model)
    gather_k = _make_sc_gather(b * s_dec, d_model, b * s_enc)
    hid, ehs_out = gather_k(embedding_weight, ids_flat, ehs_flat)
    decoder_hidden_states = hid.reshape(b, s_dec, d_model)
    ehs_out = ehs_out.reshape(encoder_hidden_states.shape)

    epb_out = encoder_position_bias

    dec_ext, enc_ext = _make_masks(decoder_attention_mask, encoder_attention_mask)

    decoder_position_bias = jnp.zeros((b, NUM_HEADS, s_dec, 1), dtype=jnp.float32)

    return (ehs_out, epb_out, decoder_hidden_states,
            enc_ext, dec_ext, decoder_position_bias)
